# baseline (speedup 1.0000x reference)
import functools
import jax
import jax.numpy as jnp
from jax import lax
from jax.experimental import pallas as pl
from jax.experimental.pallas import tpu as pltpu
from jax.experimental.pallas import tpu_sc as plsc

P = 0.15

_TC_ROWS = 80000
_TC_BLOCK = 20000

_NC = 2
_NS = 16
_NW = _NC * _NS
_CHUNK = 200
_SC_ROWS = 20000
_T = _SC_ROWS // _CHUNK
_KMAX = -(-_T // _NW)
_TAIL = _T - _NW * (_KMAX - 1)
_NBUF = 3
_L = 16


def _tc_body(mask_ref, x_ref, o_ref):
    keep = (mask_ref[...] >= P).astype(x_ref.dtype)
    o_ref[...] = x_ref[...] * keep


def _tc_mask(x, mask_u):
    n, d = x.shape
    grid = _TC_ROWS // _TC_BLOCK
    return pl.pallas_call(
        _tc_body,
        grid=(grid,),
        in_specs=[
            pl.BlockSpec((1, d), lambda i: (0, 0)),
            pl.BlockSpec((_TC_BLOCK, d), lambda i: (i, 0)),
        ],
        out_specs=pl.BlockSpec((_TC_BLOCK, d), lambda i: (i, 0)),
        out_shape=jax.ShapeDtypeStruct((_TC_ROWS, d), x.dtype),
    )(mask_u.reshape(1, d), x)


def _sc_mask(x, mask_u):
    n, d = x.shape
    mesh = plsc.VectorSubcoreMesh(core_axis_name="c", subcore_axis_name="s")

    @functools.partial(
        pl.kernel,
        out_type=jax.ShapeDtypeStruct((_SC_ROWS, d), x.dtype),
        mesh=mesh,
        scratch_types=[
            pltpu.VMEM((_CHUNK, d), jnp.float32),
            pltpu.VMEM((_CHUNK, d), jnp.float32),
            pltpu.VMEM((_CHUNK, d), jnp.float32),
            pltpu.VMEM((d,), jnp.float32),
            pltpu.SemaphoreType.DMA,
            pltpu.SemaphoreType.DMA,
            pltpu.SemaphoreType.DMA,
            pltpu.SemaphoreType.DMA,
            pltpu.SemaphoreType.DMA,
            pltpu.SemaphoreType.DMA,
        ],
    )
    def run(x_hbm, mask_hbm, out_hbm, b0, b1, b2, mask_v,
            si0, si1, si2, so0, so1, so2):
        wid = lax.axis_index("s") * _NC + lax.axis_index("c")
        bufs = (b0, b1, b2)
        isems = (si0, si1, si2)
        osems = (so0, so1, so2)

        pltpu.sync_copy(mask_hbm, mask_v)
        keep = [
            jnp.where(mask_v[pl.ds(_L * g, _L)] < P, 0.0, 1.0)
            for g in range(d // _L)
        ]

        def in_rows(k):
            return pl.ds(_TC_ROWS + (wid + _NW * k) * _CHUNK, _CHUNK)

        def out_rows(k):
            return pl.ds((wid + _NW * k) * _CHUNK, _CHUNK)

        def start_in(k):
            pltpu.async_copy(x_hbm.at[in_rows(k)], bufs[k % _NBUF],
                             isems[k % _NBUF])

        def wait_in(k):
            pltpu.make_async_copy(x_hbm.at[in_rows(k)], bufs[k % _NBUF],
                                  isems[k % _NBUF]).wait()

        def start_out(k):
            pltpu.async_copy(bufs[k % _NBUF], out_hbm.at[out_rows(k)],
                             osems[k % _NBUF])

        def wait_out(k):
            pltpu.make_async_copy(bufs[k % _NBUF], out_hbm.at[out_rows(k)],
                                  osems[k % _NBUF]).wait()

        def guarded(k, fn):
            if k < _KMAX - 1:
                fn()
            else:
                pl.when(wid < _TAIL)(fn)

        def compute(k):
            buf = bufs[k % _NBUF]

            def row_body(r, carry):
                for g in range(d // _L):
                    sl = pl.ds(_L * g, _L)
                    buf[r, sl] = buf[r, sl] * keep[g]
                return carry

            lax.fori_loop(0, _CHUNK, row_body, 0)

        guarded(0, lambda: start_in(0))
        if _KMAX > 1:
            guarded(1, lambda: start_in(1))
        for k in range(_KMAX):
            def stage(k=k):
                wait_in(k)
                compute(k)
                start_out(k)
            guarded(k, stage)
            if k + 2 < _KMAX:
                if k >= 1:
                    guarded(k - 1, lambda k=k: wait_out(k - 1))
                guarded(k + 2, lambda k=k: start_in(k + 2))
        for k in range(max(0, _KMAX - 3), _KMAX):
            guarded(k, lambda k=k: wait_out(k))

    return run(x, mask_u)


def kernel(x, y, mask_u):
    top = _tc_mask(x, mask_u)
    bottom = _sc_mask(x, mask_u)
    return (jnp.concatenate([top, bottom], axis=0), y)


# speedup vs baseline: 1.3903x; 1.3903x over previous
import functools
import jax
import jax.numpy as jnp
from jax import lax
from jax.experimental import pallas as pl
from jax.experimental.pallas import tpu as pltpu
from jax.experimental.pallas import tpu_sc as plsc

P = 0.15

_TC_ROWS = 80000
_TC_BLOCK = 20000

_NC = 2
_NS = 16
_NW = _NC * _NS
_CHUNK = 200
_SC_ROWS = 20000
_T = _SC_ROWS // _CHUNK
_KMAX = -(-_T // _NW)
_TAIL = _T - _NW * (_KMAX - 1)
_NBUF = 3
_L = 16


def _tc_body(mask_ref, x_ref, o_ref):
    keep = (mask_ref[...] >= P).astype(x_ref.dtype)
    o_ref[...] = x_ref[...] * keep


def _tc_mask(x, mask_u):
    n, d = x.shape
    grid = _TC_ROWS // _TC_BLOCK
    return pl.pallas_call(
        _tc_body,
        grid=(grid,),
        in_specs=[
            pl.BlockSpec((1, d), lambda i: (0, 0)),
            pl.BlockSpec((_TC_BLOCK, d), lambda i: (i, 0)),
        ],
        out_specs=pl.BlockSpec((_TC_BLOCK, d), lambda i: (i, 0)),
        out_shape=jax.ShapeDtypeStruct((n, d), x.dtype),
    )(mask_u.reshape(1, d), x)


def _sc_mask(x, mask_u):
    n, d = x.shape
    mesh = plsc.VectorSubcoreMesh(core_axis_name="c", subcore_axis_name="s")

    @functools.partial(
        pl.kernel,
        out_type=jax.ShapeDtypeStruct((_SC_ROWS, d), x.dtype),
        mesh=mesh,
        scratch_types=[
            pltpu.VMEM((_CHUNK, d), jnp.float32),
            pltpu.VMEM((_CHUNK, d), jnp.float32),
            pltpu.VMEM((_CHUNK, d), jnp.float32),
            pltpu.VMEM((d,), jnp.float32),
            pltpu.SemaphoreType.DMA,
            pltpu.SemaphoreType.DMA,
            pltpu.SemaphoreType.DMA,
            pltpu.SemaphoreType.DMA,
            pltpu.SemaphoreType.DMA,
            pltpu.SemaphoreType.DMA,
        ],
    )
    def run(x_hbm, mask_hbm, out_hbm, b0, b1, b2, mask_v,
            si0, si1, si2, so0, so1, so2):
        wid = lax.axis_index("s") * _NC + lax.axis_index("c")
        bufs = (b0, b1, b2)
        isems = (si0, si1, si2)
        osems = (so0, so1, so2)

        pltpu.sync_copy(mask_hbm, mask_v)
        keep = [
            jnp.where(mask_v[pl.ds(_L * g, _L)] < P, 0.0, 1.0)
            for g in range(d // _L)
        ]

        def in_rows(k):
            return pl.ds(_TC_ROWS + (wid + _NW * k) * _CHUNK, _CHUNK)

        def out_rows(k):
            return pl.ds((wid + _NW * k) * _CHUNK, _CHUNK)

        def start_in(k):
            pltpu.async_copy(x_hbm.at[in_rows(k)], bufs[k % _NBUF],
                             isems[k % _NBUF])

        def wait_in(k):
            pltpu.make_async_copy(x_hbm.at[in_rows(k)], bufs[k % _NBUF],
                                  isems[k % _NBUF]).wait()

        def start_out(k):
            pltpu.async_copy(bufs[k % _NBUF], out_hbm.at[out_rows(k)],
                             osems[k % _NBUF])

        def wait_out(k):
            pltpu.make_async_copy(bufs[k % _NBUF], out_hbm.at[out_rows(k)],
                                  osems[k % _NBUF]).wait()

        def guarded(k, fn):
            if k < _KMAX - 1:
                fn()
            else:
                pl.when(wid < _TAIL)(fn)

        def compute(k):
            buf = bufs[k % _NBUF]

            def row_body(r, carry):
                for g in range(d // _L):
                    sl = pl.ds(_L * g, _L)
                    buf[r, sl] = buf[r, sl] * keep[g]
                return carry

            lax.fori_loop(0, _CHUNK, row_body, 0)

        guarded(0, lambda: start_in(0))
        if _KMAX > 1:
            guarded(1, lambda: start_in(1))
        for k in range(_KMAX):
            def stage(k=k):
                wait_in(k)
                compute(k)
                start_out(k)
            guarded(k, stage)
            if k + 2 < _KMAX:
                if k >= 1:
                    guarded(k - 1, lambda k=k: wait_out(k - 1))
                guarded(k + 2, lambda k=k: start_in(k + 2))
        for k in range(max(0, _KMAX - 3), _KMAX):
            guarded(k, lambda k=k: wait_out(k))

    return run(x, mask_u)


def kernel(x, y, mask_u):
    full = _tc_mask(x, mask_u)
    bottom = _sc_mask(x, mask_u)
    out = lax.dynamic_update_slice(full, bottom, (_TC_ROWS, 0))
    return (out, y)


# speedup vs baseline: 1.3969x; 1.0047x over previous
import functools
import jax
import jax.numpy as jnp
from jax import lax
from jax.experimental import pallas as pl
from jax.experimental.pallas import tpu as pltpu
from jax.experimental.pallas import tpu_sc as plsc

P = 0.15

_TC_ROWS = 80000
_TC_BLOCK = 20000

_NC = 2
_NS = 16
_NW = _NC * _NS
_CHUNK = 200
_SC_ROWS = 20000
_T = _SC_ROWS // _CHUNK
_KMAX = -(-_T // _NW)
_TAIL = _T - _NW * (_KMAX - 1)
_NBUF = 3
_L = 16


def _tc_body(mask_ref, x_ref, o_ref):
    keep = (mask_ref[...] >= P).astype(x_ref.dtype)
    o_ref[...] = x_ref[...] * keep


def _tc_mask(x, mask_u):
    n, d = x.shape
    grid = _TC_ROWS // _TC_BLOCK
    return pl.pallas_call(
        _tc_body,
        grid=(grid,),
        in_specs=[
            pl.BlockSpec((1, d), lambda i: (0, 0)),
            pl.BlockSpec((_TC_BLOCK, d), lambda i: (i, 0)),
        ],
        out_specs=pl.BlockSpec((_TC_BLOCK, d), lambda i: (i, 0)),
        out_shape=jax.ShapeDtypeStruct((n, d), x.dtype),
    )(mask_u.reshape(1, d), x)


def _sc_mask(x, mask_u):
    n, d = x.shape
    mesh = plsc.VectorSubcoreMesh(core_axis_name="c", subcore_axis_name="s")

    @functools.partial(
        pl.kernel,
        out_type=jax.ShapeDtypeStruct((_SC_ROWS, d), x.dtype),
        mesh=mesh,
        scratch_types=[
            pltpu.VMEM((_CHUNK, d), jnp.float32),
            pltpu.VMEM((_CHUNK, d), jnp.float32),
            pltpu.VMEM((_CHUNK, d), jnp.float32),
            pltpu.VMEM((d,), jnp.float32),
            pltpu.SemaphoreType.DMA,
            pltpu.SemaphoreType.DMA,
            pltpu.SemaphoreType.DMA,
            pltpu.SemaphoreType.DMA,
            pltpu.SemaphoreType.DMA,
            pltpu.SemaphoreType.DMA,
        ],
    )
    def run(x_hbm, mask_hbm, out_hbm, b0, b1, b2, mask_v,
            si0, si1, si2, so0, so1, so2):
        wid = lax.axis_index("s") * _NC + lax.axis_index("c")
        bufs = (b0, b1, b2)
        isems = (si0, si1, si2)
        osems = (so0, so1, so2)

        pltpu.sync_copy(mask_hbm, mask_v)
        keep = [
            jnp.where(mask_v[pl.ds(_L * g, _L)] < P, 0.0, 1.0)
            for g in range(d // _L)
        ]

        def in_rows(k):
            return pl.ds(_TC_ROWS + (wid + _NW * k) * _CHUNK, _CHUNK)

        def out_rows(k):
            return pl.ds((wid + _NW * k) * _CHUNK, _CHUNK)

        def start_in(k):
            pltpu.async_copy(x_hbm.at[in_rows(k)], bufs[k % _NBUF],
                             isems[k % _NBUF])

        def wait_in(k):
            pltpu.make_async_copy(x_hbm.at[in_rows(k)], bufs[k % _NBUF],
                                  isems[k % _NBUF]).wait()

        def start_out(k):
            pltpu.async_copy(bufs[k % _NBUF], out_hbm.at[out_rows(k)],
                             osems[k % _NBUF])

        def wait_out(k):
            pltpu.make_async_copy(bufs[k % _NBUF], out_hbm.at[out_rows(k)],
                                  osems[k % _NBUF]).wait()

        def guarded(k, fn):
            if k < _KMAX - 1:
                fn()
            else:
                pl.when(wid < _TAIL)(fn)

        def compute(k):
            buf = bufs[k % _NBUF]

            def row_body(r, carry):
                for g in range(d // _L):
                    sl = pl.ds(_L * g, _L)
                    buf[r, sl] = buf[r, sl] * keep[g]
                return carry

            lax.fori_loop(0, _CHUNK, row_body, 0)

        guarded(0, lambda: start_in(0))
        if _KMAX > 1:
            guarded(1, lambda: start_in(1))
        for k in range(_KMAX):
            def stage(k=k):
                wait_in(k)
                compute(k)
                start_out(k)
            guarded(k, stage)
            if k + 2 < _KMAX:
                if k >= 1:
                    guarded(k - 1, lambda k=k: wait_out(k - 1))
                guarded(k + 2, lambda k=k: start_in(k + 2))
        for k in range(max(0, _KMAX - 3), _KMAX):
            guarded(k, lambda k=k: wait_out(k))

    return run(x, mask_u)


def kernel(x, y, mask_u):
    bottom = _sc_mask(x, mask_u)
    full = _tc_mask(x, mask_u)
    out = lax.dynamic_update_slice(full, bottom, (_TC_ROWS, 0))
    return (out, y)


# speedup vs baseline: 2.2274x; 1.5945x over previous
import jax
import jax.numpy as jnp
from jax.experimental import pallas as pl
from jax.experimental.pallas import tpu as pltpu

P = 0.15

_CHUNK = 2500
_NCHUNKS = 40
_NBUF = 4
_LOOKAHEAD = 3


def _body(x_hbm, mask_ref, o_hbm, b0, b1, b2, b3,
          si0, si1, si2, si3, so0, so1, so2, so3):
    bufs = (b0, b1, b2, b3)
    isems = (si0, si1, si2, si3)
    osems = (so0, so1, so2, so3)

    keep = (mask_ref[...] >= P).astype(jnp.float32)

    def rows(k):
        return pl.ds(k * _CHUNK, _CHUNK)

    def in_copy(k):
        return pltpu.make_async_copy(x_hbm.at[rows(k)], bufs[k % _NBUF],
                                     isems[k % _NBUF])

    def out_copy(k):
        return pltpu.make_async_copy(bufs[k % _NBUF], o_hbm.at[rows(k)],
                                     osems[k % _NBUF])

    for k in range(_LOOKAHEAD):
        in_copy(k).start()
    for k in range(_NCHUNKS):
        in_copy(k).wait()
        buf = bufs[k % _NBUF]
        buf[...] = buf[...] * keep
        out_copy(k).start()
        if k + _LOOKAHEAD < _NCHUNKS:
            if k + _LOOKAHEAD >= _NBUF:
                out_copy(k + _LOOKAHEAD - _NBUF).wait()
            in_copy(k + _LOOKAHEAD).start()
    for k in range(_NCHUNKS - _NBUF, _NCHUNKS):
        out_copy(k).wait()


def kernel(x, y, mask_u):
    n, d = x.shape
    x_masked = pl.pallas_call(
        _body,
        grid=(1,),
        in_specs=[
            pl.BlockSpec(memory_space=pltpu.MemorySpace.HBM),
            pl.BlockSpec((1, d), lambda i: (0, 0)),
        ],
        out_specs=pl.BlockSpec(memory_space=pltpu.MemorySpace.HBM),
        out_shape=jax.ShapeDtypeStruct((n, d), x.dtype),
        scratch_shapes=(
            [pltpu.VMEM((_CHUNK, d), jnp.float32) for _ in range(_NBUF)]
            + [pltpu.SemaphoreType.DMA] * (2 * _NBUF)
        ),
    )(x, mask_u.reshape(1, d))
    return (x_masked, y)


# speedup vs baseline: 2.4340x; 1.0928x over previous
import jax
import jax.numpy as jnp
from jax.experimental import pallas as pl
from jax.experimental.pallas import tpu as pltpu

P = 0.15

_CHUNK = 10000
_NCHUNKS = 10
_NBUF = 4
_LOOKAHEAD = 3


def _body(x_hbm, mask_ref, o_hbm, b0, b1, b2, b3,
          si0, si1, si2, si3, so0, so1, so2, so3):
    bufs = (b0, b1, b2, b3)
    isems = (si0, si1, si2, si3)
    osems = (so0, so1, so2, so3)

    keep = (mask_ref[...] >= P).astype(jnp.float32)

    def rows(k):
        return pl.ds(k * _CHUNK, _CHUNK)

    def in_copy(k):
        return pltpu.make_async_copy(x_hbm.at[rows(k)], bufs[k % _NBUF],
                                     isems[k % _NBUF])

    def out_copy(k):
        return pltpu.make_async_copy(bufs[k % _NBUF], o_hbm.at[rows(k)],
                                     osems[k % _NBUF])

    for k in range(_LOOKAHEAD):
        in_copy(k).start()
    for k in range(_NCHUNKS):
        in_copy(k).wait()
        buf = bufs[k % _NBUF]
        buf[...] = buf[...] * keep
        out_copy(k).start()
        if k + _LOOKAHEAD < _NCHUNKS:
            if k + _LOOKAHEAD >= _NBUF:
                out_copy(k + _LOOKAHEAD - _NBUF).wait()
            in_copy(k + _LOOKAHEAD).start()
    for k in range(_NCHUNKS - _NBUF, _NCHUNKS):
        out_copy(k).wait()


def kernel(x, y, mask_u):
    n, d = x.shape
    x_masked = pl.pallas_call(
        _body,
        grid=(1,),
        in_specs=[
            pl.BlockSpec(memory_space=pltpu.MemorySpace.HBM),
            pl.BlockSpec((1, d), lambda i: (0, 0)),
        ],
        out_specs=pl.BlockSpec(memory_space=pltpu.MemorySpace.HBM),
        out_shape=jax.ShapeDtypeStruct((n, d), x.dtype),
        scratch_shapes=(
            [pltpu.VMEM((_CHUNK, d), jnp.float32) for _ in range(_NBUF)]
            + [pltpu.SemaphoreType.DMA] * (2 * _NBUF)
        ),
    )(x, mask_u.reshape(1, d))
    return (x_masked, y)
